# Initial kernel scaffold; baseline (speedup 1.0000x reference)
import jax
import jax.numpy as jnp
from jax.experimental import pallas as pl


def kernel(genre, mood, artist, tempo, table_genre, table_mood, table_artist, table_tempo, W1, b1, W2, b2):
    raise NotImplementedError("write your pallas kernel here")


# speedup vs baseline: 1.3740x; 1.3740x over previous
import functools

import jax
import jax.numpy as jnp
from jax import lax
from jax.experimental import pallas as pl
from jax.experimental.pallas import tpu as pltpu
from jax.experimental.pallas import tpu_sc as plsc

B = 16384
DIM = 64
LATENT = 128
N_TABLES = 4


def _make_sc_gather():
    info = plsc.get_sparse_core_info()
    nc, ns = info.num_cores, info.num_subcores
    nw = nc * ns
    b_per_w = B // nw
    mesh = plsc.VectorSubcoreMesh(core_axis_name="c", subcore_axis_name="s")

    @functools.partial(
        pl.kernel,
        mesh=mesh,
        compiler_params=pltpu.CompilerParams(use_tc_tiling_on_sc=False),
        out_type=[jax.ShapeDtypeStruct((B, DIM), jnp.float32)] * N_TABLES,
        scratch_types=[
            pltpu.VMEM((b_per_w,), jnp.int32),
            pltpu.VMEM((b_per_w,), jnp.int32),
            pltpu.VMEM((b_per_w, DIM), jnp.float32),
            pltpu.VMEM((b_per_w, DIM), jnp.float32),
            pltpu.SemaphoreType.DMA,
            pltpu.SemaphoreType.DMA,
        ],
    )
    def gather_all(ig, im, ia, it, tg, tm, ta, tt,
                   og, om, oa, ot, idx_a, idx_b, rows_a, rows_b, sem_a, sem_b):
        wid = lax.axis_index("s") * nc + lax.axis_index("c")
        base = wid * b_per_w
        ios = ((ig, tg, og), (im, tm, om), (ia, ta, oa), (it, tt, ot))
        idxs = (idx_a, idx_b)
        rows = (rows_a, rows_b)
        sems = (sem_a, sem_b)
        copies = []
        for k, (idx_hbm, tab_hbm, out_hbm) in enumerate(ios):
            pltpu.sync_copy(idx_hbm.at[pl.ds(base, b_per_w)], idxs[k % 2])
            copies.append(
                pltpu.async_copy(tab_hbm.at[idxs[k % 2]], rows[k % 2], sems[k % 2])
            )
            if k > 0:
                copies[k - 1].wait()
                _, _, prev_out = ios[k - 1]
                pltpu.sync_copy(rows[(k - 1) % 2], prev_out.at[pl.ds(base, b_per_w)])
        copies[-1].wait()
        pltpu.sync_copy(rows[(N_TABLES - 1) % 2], ot.at[pl.ds(base, b_per_w)])

    return gather_all


_sc_gather_cache = []


def _sc_gather(*args):
    if not _sc_gather_cache:
        _sc_gather_cache.append(_make_sc_gather())
    return _sc_gather_cache[0](*args)

_BLK = 2048


def _mlp_body(eg, em, ea, et, w1, b1, w2, b2, o):
    x = (eg[...] @ w1[0 * DIM:1 * DIM, :]
         + em[...] @ w1[1 * DIM:2 * DIM, :]
         + ea[...] @ w1[2 * DIM:3 * DIM, :]
         + et[...] @ w1[3 * DIM:4 * DIM, :])
    h = jnp.maximum(x + b1[...], 0.0)
    o[...] = h @ w2[...] + b2[...]


def _mlp(eg, em, ea, et, w1, b1, w2, b2):
    grid = (B // _BLK,)
    e_spec = pl.BlockSpec((_BLK, DIM), lambda i: (i, 0))
    full = lambda shape: pl.BlockSpec(shape, lambda i: (0,) * len(shape))
    return pl.pallas_call(
        _mlp_body,
        grid=grid,
        in_specs=[e_spec, e_spec, e_spec, e_spec,
                  full((N_TABLES * DIM, LATENT)), full((1, LATENT)),
                  full((LATENT, LATENT)), full((1, LATENT))],
        out_specs=pl.BlockSpec((_BLK, LATENT), lambda i: (i, 0)),
        out_shape=jax.ShapeDtypeStruct((B, LATENT), jnp.float32),
    )(eg, em, ea, et, w1, b1, w2, b2)


def kernel(genre, mood, artist, tempo, table_genre, table_mood,
           table_artist, table_tempo, W1, b1, W2, b2):
    eg, em, ea, et = _sc_gather(
        genre.astype(jnp.int32), mood.astype(jnp.int32),
        artist.astype(jnp.int32), tempo.astype(jnp.int32),
        table_genre, table_mood, table_artist, table_tempo)
    return _mlp(eg, em, ea, et, W1, b1.reshape(1, LATENT), W2,
                b2.reshape(1, LATENT))


# speedup vs baseline: 2.5411x; 1.8494x over previous
import functools

import jax
import jax.numpy as jnp
from jax import lax
from jax.experimental import pallas as pl
from jax.experimental.pallas import tpu as pltpu
from jax.experimental.pallas import tpu_sc as plsc

B = 16384
VOCAB = 100000
DIM = 64
LATENT = 128
N_TABLES = 4
CHUNK = 4096


def _make_sc_gather():
    info = plsc.get_sparse_core_info()
    nc, ns = info.num_cores, info.num_subcores
    nw = nc * ns
    cols_per_w = DIM // nw
    mesh = plsc.VectorSubcoreMesh(core_axis_name="c", subcore_axis_name="s")

    @functools.partial(
        pl.kernel,
        mesh=mesh,
        compiler_params=pltpu.CompilerParams(needs_layout_passes=False),
        out_type=[jax.ShapeDtypeStruct((DIM, B), jnp.float32)] * N_TABLES,
        scratch_types=[
            pltpu.VMEM((VOCAB,), jnp.float32),
            pltpu.VMEM((B,), jnp.int32),
            pltpu.VMEM((CHUNK,), jnp.float32),
            pltpu.VMEM((CHUNK,), jnp.float32),
            pltpu.SemaphoreType.DMA,
            pltpu.SemaphoreType.DMA,
        ],
    )
    def gather_all(ig, im, ia, it, tg, tm, ta, tt,
                   og, om, oa, ot, col_v, idx_v, out_a, out_b, sem_a, sem_b):
        wid = lax.axis_index("s") * nc + lax.axis_index("c")
        outs = (out_a, out_b)
        sems = (sem_a, sem_b)
        pending = [None, None]

        def gather_column(col_ref, idx_ref, out_hbm_row, c):
            for ch in range(B // CHUNK):
                buf = (c * (B // CHUNK) + ch) % 2
                if pending[buf] is not None:
                    pending[buf].wait()
                    pending[buf] = None
                out_v = outs[buf]

                def body(i, _):
                    vidx = idx_ref[pl.ds(ch * CHUNK + i * 16, 16)]
                    out_v[pl.ds(i * 16, 16)] = plsc.load_gather(
                        col_ref, [vidx])
                    return 0

                lax.fori_loop(0, CHUNK // 16, body, 0, unroll=8)
                cp = pltpu.make_async_copy(
                    out_v, out_hbm_row.at[pl.ds(ch * CHUNK, CHUNK)],
                    sems[buf])
                cp.start()
                pending[buf] = cp

        for idx_hbm, tab_hbm, out_hbm in ((ig, tg, og), (im, tm, om),
                                          (ia, ta, oa), (it, tt, ot)):
            pltpu.sync_copy(idx_hbm, idx_v)
            for c in range(cols_per_w):
                col = wid + c * nw
                pltpu.sync_copy(tab_hbm.at[col], col_v)
                gather_column(col_v, idx_v, out_hbm.at[col], c)
        for buf in range(2):
            if pending[buf] is not None:
                pending[buf].wait()

    return gather_all


_sc_gather_cache = []


def _sc_gather(*args):
    if not _sc_gather_cache:
        _sc_gather_cache.append(_make_sc_gather())
    return _sc_gather_cache[0](*args)


_BLK = 2048


def _mlp_body(eg, em, ea, et, w1, b1, w2, b2, o):
    dn = (((0,), (0,)), ((), ()))
    x = (lax.dot_general(eg[...], w1[0 * DIM:1 * DIM, :], dn)
         + lax.dot_general(em[...], w1[1 * DIM:2 * DIM, :], dn)
         + lax.dot_general(ea[...], w1[2 * DIM:3 * DIM, :], dn)
         + lax.dot_general(et[...], w1[3 * DIM:4 * DIM, :], dn))
    h = jnp.maximum(x + b1[...], 0.0)
    o[...] = h @ w2[...] + b2[...]


def _mlp(eg, em, ea, et, w1, b1, w2, b2):
    grid = (B // _BLK,)
    e_spec = pl.BlockSpec((DIM, _BLK), lambda i: (0, i))
    full = lambda shape: pl.BlockSpec(shape, lambda i: (0,) * len(shape))
    return pl.pallas_call(
        _mlp_body,
        grid=grid,
        in_specs=[e_spec, e_spec, e_spec, e_spec,
                  full((N_TABLES * DIM, LATENT)), full((1, LATENT)),
                  full((LATENT, LATENT)), full((1, LATENT))],
        out_specs=pl.BlockSpec((_BLK, LATENT), lambda i: (i, 0)),
        out_shape=jax.ShapeDtypeStruct((B, LATENT), jnp.float32),
    )(eg, em, ea, et, w1, b1, w2, b2)


def kernel(genre, mood, artist, tempo, table_genre, table_mood,
           table_artist, table_tempo, W1, b1, W2, b2):
    eg, em, ea, et = _sc_gather(
        genre.astype(jnp.int32), mood.astype(jnp.int32),
        artist.astype(jnp.int32), tempo.astype(jnp.int32),
        table_genre.T, table_mood.T, table_artist.T, table_tempo.T)
    return _mlp(eg, em, ea, et, W1, b1.reshape(1, LATENT), W2,
                b2.reshape(1, LATENT))


# speedup vs baseline: 4.0016x; 1.5747x over previous
import functools

import jax
import jax.numpy as jnp
from jax import lax
from jax.experimental import pallas as pl
from jax.experimental.pallas import tpu as pltpu
from jax.experimental.pallas import tpu_sc as plsc

B = 16384
VOCAB = 100000
DIM = 64
LATENT = 128
N_TABLES = 4
CHUNK = 4096


def _make_sc_gather():
    info = plsc.get_sparse_core_info()
    nc, ns = info.num_cores, info.num_subcores
    nw = nc * ns
    cols_per_w = DIM // nw
    mesh = plsc.VectorSubcoreMesh(core_axis_name="c", subcore_axis_name="s")

    @functools.partial(
        pl.kernel,
        mesh=mesh,
        compiler_params=pltpu.CompilerParams(needs_layout_passes=False),
        out_type=[jax.ShapeDtypeStruct((DIM, B), jnp.float32)] * N_TABLES,
        scratch_types=[
            pltpu.VMEM((VOCAB,), jnp.float32),
            pltpu.VMEM((B,), jnp.int32),
            pltpu.VMEM((CHUNK,), jnp.float32),
            pltpu.VMEM((CHUNK,), jnp.float32),
            pltpu.SemaphoreType.DMA,
            pltpu.SemaphoreType.DMA,
        ],
    )
    def gather_all(ig, im, ia, it, tg, tm, ta, tt,
                   og, om, oa, ot, col_v, idx_v, out_a, out_b, sem_a, sem_b):
        wid = lax.axis_index("s") * nc + lax.axis_index("c")
        outs = (out_a, out_b)
        sems = (sem_a, sem_b)
        pending = [None, None]

        def gather_column(col_ref, idx_ref, out_hbm_row, c):
            for ch in range(B // CHUNK):
                buf = (c * (B // CHUNK) + ch) % 2
                if pending[buf] is not None:
                    pending[buf].wait()
                    pending[buf] = None
                out_v = outs[buf]

                @plsc.parallel_loop(0, CHUNK // 16, unroll=8)
                def body(i):
                    vidx = idx_ref[pl.ds(ch * CHUNK + i * 16, 16)]
                    out_v[pl.ds(i * 16, 16)] = plsc.load_gather(
                        col_ref, [vidx])
                cp = pltpu.make_async_copy(
                    out_v, out_hbm_row.at[pl.ds(ch * CHUNK, CHUNK)],
                    sems[buf])
                cp.start()
                pending[buf] = cp

        for idx_hbm, tab_hbm, out_hbm in ((ig, tg, og), (im, tm, om),
                                          (ia, ta, oa), (it, tt, ot)):
            pltpu.sync_copy(idx_hbm, idx_v)
            for c in range(cols_per_w):
                col = wid + c * nw
                pltpu.sync_copy(tab_hbm.at[col], col_v)
                gather_column(col_v, idx_v, out_hbm.at[col], c)
        for buf in range(2):
            if pending[buf] is not None:
                pending[buf].wait()

    return gather_all


_sc_gather_cache = []


def _sc_gather(*args):
    if not _sc_gather_cache:
        _sc_gather_cache.append(_make_sc_gather())
    return _sc_gather_cache[0](*args)


_BLK = 2048


def _mlp_body(eg, em, ea, et, w1, b1, w2, b2, o):
    dn = (((0,), (0,)), ((), ()))
    x = (lax.dot_general(eg[...], w1[0 * DIM:1 * DIM, :], dn)
         + lax.dot_general(em[...], w1[1 * DIM:2 * DIM, :], dn)
         + lax.dot_general(ea[...], w1[2 * DIM:3 * DIM, :], dn)
         + lax.dot_general(et[...], w1[3 * DIM:4 * DIM, :], dn))
    h = jnp.maximum(x + b1[...], 0.0)
    o[...] = h @ w2[...] + b2[...]


def _mlp(eg, em, ea, et, w1, b1, w2, b2):
    grid = (B // _BLK,)
    e_spec = pl.BlockSpec((DIM, _BLK), lambda i: (0, i))
    full = lambda shape: pl.BlockSpec(shape, lambda i: (0,) * len(shape))
    return pl.pallas_call(
        _mlp_body,
        grid=grid,
        in_specs=[e_spec, e_spec, e_spec, e_spec,
                  full((N_TABLES * DIM, LATENT)), full((1, LATENT)),
                  full((LATENT, LATENT)), full((1, LATENT))],
        out_specs=pl.BlockSpec((_BLK, LATENT), lambda i: (i, 0)),
        out_shape=jax.ShapeDtypeStruct((B, LATENT), jnp.float32),
    )(eg, em, ea, et, w1, b1, w2, b2)


def kernel(genre, mood, artist, tempo, table_genre, table_mood,
           table_artist, table_tempo, W1, b1, W2, b2):
    eg, em, ea, et = _sc_gather(
        genre.astype(jnp.int32), mood.astype(jnp.int32),
        artist.astype(jnp.int32), tempo.astype(jnp.int32),
        table_genre.T, table_mood.T, table_artist.T, table_tempo.T)
    return _mlp(eg, em, ea, et, W1, b1.reshape(1, LATENT), W2,
                b2.reshape(1, LATENT))
